# Initial kernel scaffold; baseline (speedup 1.0000x reference)
import jax
import jax.numpy as jnp
from jax.experimental import pallas as pl


def kernel(x, nodes, nbr1, nbr2, W_self0, W_neigh0, b0, W_self1, W_neigh1, b1, Wp, bp):
    raise NotImplementedError("write your pallas kernel here")


# speedup vs baseline: 1.9503x; 1.9503x over previous
import functools

import jax
import jax.numpy as jnp
from jax import lax
from jax.experimental import pallas as pl
from jax.experimental.pallas import tpu as pltpu
from jax.experimental.pallas import tpu_sc as plsc

T = 3
N = 100000
D = 128
B = 4096
S1, S2 = 5, 2
H0, H1 = 64, 32
OUT = 64
V_TH = 1.0

NC, NS = 2, 16
NW = NC * NS
CH = 128

A0_TASKS = (T * B) // (NW * CH)
A1_TASKS = (T * B) // (NW * CH)
N1_TASKS = (T * S1 * B) // (NW * CH)


def _gather_body(xf, idxA0, idxA1, idxN2, A0, A1, Nm0, Nm1,
                 buf, obuf, idx2d, gsem):
    wid = lax.axis_index("s") * NC + lax.axis_index("c")

    def a0_task(ci, _):
        base = pl.multiple_of((ci * NW + wid) * CH, CH)
        pltpu.sync_copy(idxA0.at[pl.ds(base, CH)], idx2d.at[0])
        pltpu.async_copy(xf.at[idx2d.at[0]], buf.at[pl.ds(0, CH)], gsem).wait()
        pltpu.sync_copy(buf.at[pl.ds(0, CH)], A0.at[pl.ds(base, CH)])
        return 0

    lax.fori_loop(0, A0_TASKS, a0_task, 0)

    def a1_task(ci, _):
        t = ci
        sbase = wid * CH
        for k in range(S1):
            src = pl.multiple_of(t * S1 * B + k * B + sbase, CH)
            pltpu.sync_copy(idxA1.at[pl.ds(src, CH)], idx2d.at[k])
        cps = [pltpu.async_copy(xf.at[idx2d.at[k]],
                                buf.at[pl.ds(k * CH, CH)], gsem)
               for k in range(S1)]
        for cp in cps:
            cp.wait()
        for k in range(S1):
            dst = pl.multiple_of(t * S1 * B + k * B + sbase, CH)
            pltpu.sync_copy(buf.at[pl.ds(k * CH, CH)], A1.at[pl.ds(dst, CH)])

        def mean5(j, _):
            for c in range(D // 16):
                s = pl.ds(c * 16, 16)
                acc = (buf[j, s] + buf[CH + j, s] + buf[2 * CH + j, s]
                       + buf[3 * CH + j, s] + buf[4 * CH + j, s])
                obuf[j, s] = acc * (1.0 / S1)
            return 0

        lax.fori_loop(0, CH, mean5, 0)
        pltpu.sync_copy(obuf, Nm0.at[pl.ds(pl.multiple_of(t * B + sbase, CH), CH)])
        return 0

    lax.fori_loop(0, A1_TASKS, a1_task, 0)

    def n1_task(ci, _):
        t = ci // S1
        k = ci % S1
        row0 = t * S1 * B + k * B + wid * CH
        src = pl.multiple_of(row0 * S2, CH)
        pltpu.sync_copy(idxN2.at[pl.ds(src, CH)], idx2d.at[0])
        pltpu.sync_copy(idxN2.at[pl.ds(src + CH, CH)], idx2d.at[1])
        cps = [pltpu.async_copy(xf.at[idx2d.at[p]],
                                buf.at[pl.ds(p * CH, CH)], gsem)
               for p in range(S2)]
        for cp in cps:
            cp.wait()

        def mean2(j, _):
            for c in range(D // 16):
                s = pl.ds(c * 16, 16)
                obuf[j, s] = (buf[2 * j, s] + buf[2 * j + 1, s]) * (1.0 / S2)
            return 0

        lax.fori_loop(0, CH, mean2, 0)
        pltpu.sync_copy(obuf, Nm1.at[pl.ds(pl.multiple_of(row0, CH), CH)])
        return 0

    lax.fori_loop(0, N1_TASKS, n1_task, 0)


@functools.lru_cache(maxsize=1)
def _make_gather_call():
    return functools.partial(
        pl.kernel,
        out_type=[
            jax.ShapeDtypeStruct((T * B, D), jnp.float32),
            jax.ShapeDtypeStruct((T * S1 * B, D), jnp.float32),
            jax.ShapeDtypeStruct((T * B, D), jnp.float32),
            jax.ShapeDtypeStruct((T * S1 * B, D), jnp.float32),
        ],
        mesh=plsc.VectorSubcoreMesh(core_axis_name="c", subcore_axis_name="s",
                                    num_cores=NC, num_subcores=NS),
        scratch_types=[
            pltpu.VMEM((S1 * CH, D), jnp.float32),
            pltpu.VMEM((CH, D), jnp.float32),
            pltpu.VMEM((S1, CH), jnp.int32),
            pltpu.SemaphoreType.DMA,
        ],
    )(_gather_body)


def _dense_body(a0, a1, m0, m1, ws0, wn0, b0, ws1, wn1, b1, wp, bp, out):
    f32 = jnp.float32
    s1s = []
    for t in range(T):
        p0 = (jnp.dot(a0[t], ws0[...], preferred_element_type=f32)
              + jnp.dot(m0[t], wn0[...], preferred_element_type=f32) + b0[...])
        g0 = (p0 >= V_TH).astype(f32)
        nn = jnp.zeros((a0.shape[1], H0), f32)
        for k in range(S1):
            p1 = (jnp.dot(a1[t, k], ws0[...], preferred_element_type=f32)
                  + jnp.dot(m1[t, k], wn0[...], preferred_element_type=f32)
                  + b0[...])
            nn = nn + (p1 >= V_TH).astype(f32)
        q = (jnp.dot(g0, ws1[...], preferred_element_type=f32)
             + jnp.dot(nn * (1.0 / S1), wn1[...], preferred_element_type=f32)
             + b1[...])
        s1s.append((q >= V_TH).astype(f32))
    sp = jnp.concatenate(s1s, axis=1)
    out[...] = jnp.dot(sp, wp[...], preferred_element_type=f32) + bp[...]


def _dense_stage(a0, a1, m0, m1, ws0, wn0, b0, ws1, wn1, b1, wp, bp):
    blk = 256
    grid = (B // blk,)
    full = lambda shape: pl.BlockSpec(shape, lambda i: (0,) * len(shape))
    return pl.pallas_call(
        _dense_body,
        grid=grid,
        in_specs=[
            pl.BlockSpec((T, blk, D), lambda i: (0, i, 0)),
            pl.BlockSpec((T, S1, blk, D), lambda i: (0, 0, i, 0)),
            pl.BlockSpec((T, blk, D), lambda i: (0, i, 0)),
            pl.BlockSpec((T, S1, blk, D), lambda i: (0, 0, i, 0)),
            full((D, H0)), full((D, H0)), full((1, H0)),
            full((H0, H1)), full((H0, H1)), full((1, H1)),
            full((T * H1, OUT)), full((1, OUT)),
        ],
        out_specs=pl.BlockSpec((blk, OUT), lambda i: (i, 0)),
        out_shape=jax.ShapeDtypeStruct((B, OUT), jnp.float32),
    )(a0, a1, m0, m1, ws0, wn0, b0, ws1, wn1, b1, wp, bp)


def kernel(x, nodes, nbr1, nbr2, W_self0, W_neigh0, b0, W_self1, W_neigh1,
           b1, Wp, bp):
    xf = x.reshape(T * N, D)
    i32 = jnp.int32
    toff = (jnp.arange(T, dtype=i32) * N)
    idxA0 = (nodes.astype(i32)[None, :] + toff[:, None]).reshape(-1)
    idxA1 = (nbr1.astype(i32).reshape(T, B, S1).transpose(0, 2, 1)
             + toff[:, None, None]).reshape(-1)
    idxN2 = (nbr2.astype(i32).reshape(T, B, S1, S2).transpose(0, 2, 1, 3)
             + toff[:, None, None, None]).reshape(-1)

    A0, A1, Nm0, Nm1 = _make_gather_call()(xf, idxA0, idxA1, idxN2)

    return _dense_stage(
        A0.reshape(T, B, D), A1.reshape(T, S1, B, D),
        Nm0.reshape(T, B, D), Nm1.reshape(T, S1, B, D),
        W_self0, W_neigh0, b0.reshape(1, H0),
        W_self1, W_neigh1, b1.reshape(1, H1),
        Wp, bp.reshape(1, OUT))


# speedup vs baseline: 1.9526x; 1.0012x over previous
import functools

import jax
import jax.numpy as jnp
from jax import lax
from jax.experimental import pallas as pl
from jax.experimental.pallas import tpu as pltpu
from jax.experimental.pallas import tpu_sc as plsc

T = 3
N = 100000
D = 128
B = 4096
S1, S2 = 5, 2
H0, H1 = 64, 32
OUT = 64
V_TH = 1.0

NC, NS = 2, 16
NW = NC * NS
CH = 128

A0_TASKS = (T * B) // (NW * CH)
A1_TASKS = (T * B) // (NW * CH)
N1_TASKS = (T * S1 * B) // (NW * CH)


def _gather_body(xf, idxA0, idxA1, idxN2, A0, A1, Nm0, Nm1,
                 buf, obuf, idx2d, gsem):
    wid = lax.axis_index("s") * NC + lax.axis_index("c")

    def a0_task(ci, _):
        base = pl.multiple_of((ci * NW + wid) * CH, CH)
        pltpu.sync_copy(idxA0.at[pl.ds(base, CH)], idx2d.at[0])
        pltpu.async_copy(xf.at[idx2d.at[0]], buf.at[pl.ds(0, CH)], gsem).wait()
        pltpu.sync_copy(buf.at[pl.ds(0, CH)], A0.at[pl.ds(base, CH)])
        return 0

    lax.fori_loop(0, A0_TASKS, a0_task, 0)

    def a1_task(ci, _):
        t = ci
        sbase = wid * CH
        for k in range(S1):
            src = pl.multiple_of(t * S1 * B + k * B + sbase, CH)
            pltpu.sync_copy(idxA1.at[pl.ds(src, CH)], idx2d.at[k])
        cps = [pltpu.async_copy(xf.at[idx2d.at[k]],
                                buf.at[pl.ds(k * CH, CH)], gsem)
               for k in range(S1)]
        for cp in cps:
            cp.wait()
        for k in range(S1):
            dst = pl.multiple_of(t * S1 * B + k * B + sbase, CH)
            pltpu.sync_copy(buf.at[pl.ds(k * CH, CH)], A1.at[pl.ds(dst, CH)])

        def mean5(j, _):
            for c in range(D // 16):
                s = pl.ds(c * 16, 16)
                acc = (buf[j, s] + buf[CH + j, s] + buf[2 * CH + j, s]
                       + buf[3 * CH + j, s] + buf[4 * CH + j, s])
                obuf[j, s] = acc * (1.0 / S1)
            return 0

        lax.fori_loop(0, CH, mean5, 0)
        pltpu.sync_copy(obuf, Nm0.at[pl.ds(pl.multiple_of(t * B + sbase, CH), CH)])
        return 0

    lax.fori_loop(0, A1_TASKS, a1_task, 0)

    def n1_task(ci, _):
        t = ci // S1
        k = ci % S1
        row0 = t * S1 * B + k * B + wid * CH
        src = pl.multiple_of(row0 * S2, CH)
        pltpu.sync_copy(idxN2.at[pl.ds(src, CH)], idx2d.at[0])
        pltpu.sync_copy(idxN2.at[pl.ds(src + CH, CH)], idx2d.at[1])
        cps = [pltpu.async_copy(xf.at[idx2d.at[p]],
                                buf.at[pl.ds(p * CH, CH)], gsem)
               for p in range(S2)]
        for cp in cps:
            cp.wait()

        def mean2(j, _):
            for c in range(D // 16):
                s = pl.ds(c * 16, 16)
                obuf[j, s] = (buf[2 * j, s] + buf[2 * j + 1, s]) * (1.0 / S2)
            return 0

        lax.fori_loop(0, CH, mean2, 0)
        pltpu.sync_copy(obuf, Nm1.at[pl.ds(pl.multiple_of(row0, CH), CH)])
        return 0

    lax.fori_loop(0, N1_TASKS, n1_task, 0)


@functools.lru_cache(maxsize=1)
def _make_gather_call():
    return functools.partial(
        pl.kernel,
        out_type=[
            jax.ShapeDtypeStruct((T * B, D), jnp.float32),
            jax.ShapeDtypeStruct((T * S1 * B, D), jnp.float32),
            jax.ShapeDtypeStruct((T * B, D), jnp.float32),
            jax.ShapeDtypeStruct((T * S1 * B, D), jnp.float32),
        ],
        mesh=plsc.VectorSubcoreMesh(core_axis_name="c", subcore_axis_name="s",
                                    num_cores=NC, num_subcores=NS),
        compiler_params=pltpu.CompilerParams(use_tc_tiling_on_sc=True),
        scratch_types=[
            pltpu.VMEM((S1 * CH, D), jnp.float32),
            pltpu.VMEM((CH, D), jnp.float32),
            pltpu.VMEM((S1, CH), jnp.int32),
            pltpu.SemaphoreType.DMA,
        ],
    )(_gather_body)


def _dense_body(a0, a1, m0, m1, ws0, wn0, b0, ws1, wn1, b1, wp, bp, out):
    f32 = jnp.float32
    s1s = []
    for t in range(T):
        p0 = (jnp.dot(a0[t], ws0[...], preferred_element_type=f32)
              + jnp.dot(m0[t], wn0[...], preferred_element_type=f32) + b0[...])
        g0 = (p0 >= V_TH).astype(f32)
        nn = jnp.zeros((a0.shape[1], H0), f32)
        for k in range(S1):
            p1 = (jnp.dot(a1[t, k], ws0[...], preferred_element_type=f32)
                  + jnp.dot(m1[t, k], wn0[...], preferred_element_type=f32)
                  + b0[...])
            nn = nn + (p1 >= V_TH).astype(f32)
        q = (jnp.dot(g0, ws1[...], preferred_element_type=f32)
             + jnp.dot(nn * (1.0 / S1), wn1[...], preferred_element_type=f32)
             + b1[...])
        s1s.append((q >= V_TH).astype(f32))
    sp = jnp.concatenate(s1s, axis=1)
    out[...] = jnp.dot(sp, wp[...], preferred_element_type=f32) + bp[...]


def _dense_stage(a0, a1, m0, m1, ws0, wn0, b0, ws1, wn1, b1, wp, bp):
    blk = 256
    grid = (B // blk,)
    full = lambda shape: pl.BlockSpec(shape, lambda i: (0,) * len(shape))
    return pl.pallas_call(
        _dense_body,
        grid=grid,
        in_specs=[
            pl.BlockSpec((T, blk, D), lambda i: (0, i, 0)),
            pl.BlockSpec((T, S1, blk, D), lambda i: (0, 0, i, 0)),
            pl.BlockSpec((T, blk, D), lambda i: (0, i, 0)),
            pl.BlockSpec((T, S1, blk, D), lambda i: (0, 0, i, 0)),
            full((D, H0)), full((D, H0)), full((1, H0)),
            full((H0, H1)), full((H0, H1)), full((1, H1)),
            full((T * H1, OUT)), full((1, OUT)),
        ],
        out_specs=pl.BlockSpec((blk, OUT), lambda i: (i, 0)),
        out_shape=jax.ShapeDtypeStruct((B, OUT), jnp.float32),
    )(a0, a1, m0, m1, ws0, wn0, b0, ws1, wn1, b1, wp, bp)


def kernel(x, nodes, nbr1, nbr2, W_self0, W_neigh0, b0, W_self1, W_neigh1,
           b1, Wp, bp):
    xf = x.reshape(T * N, D)
    i32 = jnp.int32
    toff = (jnp.arange(T, dtype=i32) * N)
    idxA0 = (nodes.astype(i32)[None, :] + toff[:, None]).reshape(-1)
    idxA1 = (nbr1.astype(i32).reshape(T, B, S1).transpose(0, 2, 1)
             + toff[:, None, None]).reshape(-1)
    idxN2 = (nbr2.astype(i32).reshape(T, B, S1, S2).transpose(0, 2, 1, 3)
             + toff[:, None, None, None]).reshape(-1)

    A0, A1, Nm0, Nm1 = _make_gather_call()(xf, idxA0, idxA1, idxN2)

    return _dense_stage(
        A0.reshape(T, B, D), A1.reshape(T, S1, B, D),
        Nm0.reshape(T, B, D), Nm1.reshape(T, S1, B, D),
        W_self0, W_neigh0, b0.reshape(1, H0),
        W_self1, W_neigh1, b1.reshape(1, H1),
        Wp, bp.reshape(1, OUT))


# speedup vs baseline: 3.5942x; 1.8407x over previous
import functools

import jax
import jax.numpy as jnp
from jax import lax
from jax.experimental import pallas as pl
from jax.experimental.pallas import tpu as pltpu
from jax.experimental.pallas import tpu_sc as plsc

T = 3
N = 100000
D = 128
B = 4096
S1, S2 = 5, 2
H0, H1 = 64, 32
OUT = 64
V_TH = 1.0

NC, NS = 2, 16
NW = NC * NS
CH = 128

A0_TASKS = (T * B) // (NW * CH)
A1_TASKS = (T * B) // (NW * CH)
N1_TASKS = (T * S1 * B) // (NW * CH)


def _gather_body(x, nodes, nbr1, nbr2, A0, A1, Nm0, Nm1,
                 buf, obuf, idx2d, raw, gsem):
    wid = lax.axis_index("s") * NC + lax.axis_index("c")
    lanes = jnp.arange(16, dtype=jnp.int32)

    def a0_task(ci, _):
        chunk = ci * NW + wid
        t = chunk // (B // CH)
        sbase = pl.multiple_of((chunk % (B // CH)) * CH, CH)
        pltpu.sync_copy(nodes.at[pl.ds(sbase, CH)], idx2d.at[0])
        pltpu.async_copy(x.at[t].at[idx2d.at[0]], buf.at[pl.ds(0, CH)],
                         gsem).wait()
        pltpu.sync_copy(buf.at[pl.ds(0, CH)], A0.at[t, pl.ds(sbase, CH)])
        return 0

    lax.fori_loop(0, A0_TASKS, a0_task, 0)

    def a1_task(ci, _):
        t = ci
        sbase = wid * CH
        pltpu.sync_copy(nbr1.at[t, pl.ds(pl.multiple_of(sbase * S1, CH),
                                         CH * S1)],
                        raw.at[pl.ds(0, CH * S1)])
        for k in range(S1):
            for g in range(CH // 16):
                iv = lanes * S1 + (g * 16 * S1 + k)
                vals = plsc.load_gather(raw, [iv])
                idx2d[k, pl.ds(g * 16, 16)] = vals
        cps = [pltpu.async_copy(x.at[t].at[idx2d.at[k]],
                                buf.at[pl.ds(k * CH, CH)], gsem)
               for k in range(S1)]
        for cp in cps:
            cp.wait()
        for k in range(S1):
            pltpu.sync_copy(buf.at[pl.ds(k * CH, CH)],
                            A1.at[t, k, pl.ds(sbase, CH)])

        def mean5(j, _):
            for c in range(D // 16):
                s = pl.ds(c * 16, 16)
                acc = (buf[j, s] + buf[CH + j, s] + buf[2 * CH + j, s]
                       + buf[3 * CH + j, s] + buf[4 * CH + j, s])
                obuf[j, s] = acc * (1.0 / S1)
            return 0

        lax.fori_loop(0, CH, mean5, 0)
        pltpu.sync_copy(obuf, Nm0.at[t, pl.ds(sbase, CH)])
        return 0

    lax.fori_loop(0, A1_TASKS, a1_task, 0)

    def n1_task(ci, _):
        t = ci // S1
        k = ci % S1
        sbase = wid * CH
        pltpu.sync_copy(nbr2.at[t, pl.ds(pl.multiple_of(sbase * S1 * S2, CH),
                                         CH * S1 * S2)],
                        raw.at[pl.ds(0, CH * S1 * S2)])
        for g in range(2 * CH // 16):
            iv = ((lanes // 2) * (S1 * S2) + (lanes % 2)
                  + (g * 8 * S1 * S2 + k * S2))
            vals = plsc.load_gather(raw, [iv])
            idx2d[g // 8, pl.ds((g % 8) * 16, 16)] = vals
        cps = [pltpu.async_copy(x.at[t].at[idx2d.at[p]],
                                buf.at[pl.ds(p * CH, CH)], gsem)
               for p in range(S2)]
        for cp in cps:
            cp.wait()

        def mean2(j, _):
            for c in range(D // 16):
                s = pl.ds(c * 16, 16)
                obuf[j, s] = (buf[2 * j, s] + buf[2 * j + 1, s]) * (1.0 / S2)
            return 0

        lax.fori_loop(0, CH, mean2, 0)
        pltpu.sync_copy(obuf, Nm1.at[t, k, pl.ds(sbase, CH)])
        return 0

    lax.fori_loop(0, N1_TASKS, n1_task, 0)


@functools.lru_cache(maxsize=1)
def _make_gather_call():
    return functools.partial(
        pl.kernel,
        out_type=[
            jax.ShapeDtypeStruct((T, B, D), jnp.float32),
            jax.ShapeDtypeStruct((T, S1, B, D), jnp.float32),
            jax.ShapeDtypeStruct((T, B, D), jnp.float32),
            jax.ShapeDtypeStruct((T, S1, B, D), jnp.float32),
        ],
        mesh=plsc.VectorSubcoreMesh(core_axis_name="c", subcore_axis_name="s",
                                    num_cores=NC, num_subcores=NS),
        compiler_params=pltpu.CompilerParams(use_tc_tiling_on_sc=True,
                                             needs_layout_passes=False),
        scratch_types=[
            pltpu.VMEM((S1 * CH, D), jnp.float32),
            pltpu.VMEM((CH, D), jnp.float32),
            pltpu.VMEM((S1, CH), jnp.int32),
            pltpu.VMEM((S1 * S2 * CH,), jnp.int32),
            pltpu.SemaphoreType.DMA,
        ],
    )(_gather_body)


def _dense_body(a0, a1, m0, m1, ws0, wn0, b0, ws1, wn1, b1, wp, bp, out):
    f32 = jnp.float32
    s1s = []
    for t in range(T):
        p0 = (jnp.dot(a0[t], ws0[...], preferred_element_type=f32)
              + jnp.dot(m0[t], wn0[...], preferred_element_type=f32) + b0[...])
        g0 = (p0 >= V_TH).astype(f32)
        nn = jnp.zeros((a0.shape[1], H0), f32)
        for k in range(S1):
            p1 = (jnp.dot(a1[t, k], ws0[...], preferred_element_type=f32)
                  + jnp.dot(m1[t, k], wn0[...], preferred_element_type=f32)
                  + b0[...])
            nn = nn + (p1 >= V_TH).astype(f32)
        q = (jnp.dot(g0, ws1[...], preferred_element_type=f32)
             + jnp.dot(nn * (1.0 / S1), wn1[...], preferred_element_type=f32)
             + b1[...])
        s1s.append((q >= V_TH).astype(f32))
    sp = jnp.concatenate(s1s, axis=1)
    out[...] = jnp.dot(sp, wp[...], preferred_element_type=f32) + bp[...]


def _dense_stage(a0, a1, m0, m1, ws0, wn0, b0, ws1, wn1, b1, wp, bp):
    blk = 256
    grid = (B // blk,)
    full = lambda shape: pl.BlockSpec(shape, lambda i: (0,) * len(shape))
    return pl.pallas_call(
        _dense_body,
        grid=grid,
        in_specs=[
            pl.BlockSpec((T, blk, D), lambda i: (0, i, 0)),
            pl.BlockSpec((T, S1, blk, D), lambda i: (0, 0, i, 0)),
            pl.BlockSpec((T, blk, D), lambda i: (0, i, 0)),
            pl.BlockSpec((T, S1, blk, D), lambda i: (0, 0, i, 0)),
            full((D, H0)), full((D, H0)), full((1, H0)),
            full((H0, H1)), full((H0, H1)), full((1, H1)),
            full((T * H1, OUT)), full((1, OUT)),
        ],
        out_specs=pl.BlockSpec((blk, OUT), lambda i: (i, 0)),
        out_shape=jax.ShapeDtypeStruct((B, OUT), jnp.float32),
    )(a0, a1, m0, m1, ws0, wn0, b0, ws1, wn1, b1, wp, bp)


def kernel(x, nodes, nbr1, nbr2, W_self0, W_neigh0, b0, W_self1, W_neigh1,
           b1, Wp, bp):
    A0, A1, Nm0, Nm1 = _make_gather_call()(
        x, nodes.astype(jnp.int32), nbr1.astype(jnp.int32),
        nbr2.astype(jnp.int32))

    return _dense_stage(
        A0, A1, Nm0, Nm1,
        W_self0, W_neigh0, b0.reshape(1, H0),
        W_self1, W_neigh1, b1.reshape(1, H1),
        Wp, bp.reshape(1, OUT))


# speedup vs baseline: 4.0423x; 1.1247x over previous
import functools

import jax
import jax.numpy as jnp
from jax import lax
from jax.experimental import pallas as pl
from jax.experimental.pallas import tpu as pltpu
from jax.experimental.pallas import tpu_sc as plsc

T = 3
N = 100000
D = 128
B = 4096
S1, S2 = 5, 2
H0, H1 = 64, 32
OUT = 64
V_TH = 1.0

NC, NS = 2, 16
NW = NC * NS
CH = 128

A0_TASKS = (T * B) // (NW * CH)
A1_TASKS = (T * B) // (NW * CH)
N1_TASKS = (T * S1 * B) // (NW * CH)


def _gather_body(x, nodes, nbr1, nbr2, A0, A1, Nm0, Nm1,
                 buf, obuf, ip, w1, w2, gsem, wsem, nsem):
    wid = lax.axis_index("s") * NC + lax.axis_index("c")
    lanes = jnp.arange(16, dtype=jnp.int32)
    G = CH // 16
    sbase = wid * CH

    def slot(s, n=CH):
        return buf.at[pl.ds(s * CH, n)]

    pltpu.sync_copy(nodes.at[pl.ds(sbase, CH)], ip.at[0])
    gA0 = [pltpu.async_copy(x.at[t].at[ip.at[0]], slot(t), gsem.at[t])
           for t in range(T)]
    def wcopy(t, _):
        pltpu.sync_copy(nbr1.at[t, pl.ds(sbase * S1, CH * S1)],
                        w1.at[pl.ds(t * CH * S1, CH * S1)])
        pltpu.sync_copy(nbr2.at[t, pl.ds(sbase * S1 * S2, CH * S1 * S2)],
                        w2.at[pl.ds(t * CH * S1 * S2, CH * S1 * S2)])
        return 0

    lax.fori_loop(0, T, wcopy, 0)

    def ex1(m, _):
        q = m // G
        t, k, g = q // S1, q % S1, m % G
        iv = lanes * S1 + (t * CH * S1 + g * 16 * S1 + k)
        ip[1 + q, pl.ds((m % G) * 16, 16)] = plsc.load_gather(w1, [iv])
        return 0

    lax.fori_loop(0, T * S1 * G, ex1, 0)

    def ex2(m, _):
        q = m // G
        r, g = q % (2 * S1), m % G
        t, k, h = q // (2 * S1), r // 2, r % 2
        iv = ((lanes // 2) * (S1 * S2) + (lanes % 2)
              + (t * CH * S1 * S2 + h * (CH * S1 * S2 // 2)
                 + g * 8 * S1 * S2 + k * S2))
        ip[16 + q, pl.ds(g * 16, 16)] = plsc.load_gather(w2, [iv])
        return 0

    lax.fori_loop(0, T * 2 * S1 * G, ex2, 0)

    wA0 = []
    for t in range(T):
        gA0[t].wait()
        wA0.append(pltpu.async_copy(slot(t), A0.at[t, pl.ds(sbase, CH)],
                                    wsem.at[t]))

    wNm = None
    for t in range(T):
        if t == 0:
            for cp in wA0:
                cp.wait()
        cps = [pltpu.async_copy(x.at[t].at[ip.at[1 + t * S1 + k]],
                                slot(k), gsem.at[k]) for k in range(S1)]
        ws = []
        for k in range(S1):
            cps[k].wait()
            ws.append(pltpu.async_copy(slot(k),
                                       A1.at[t, k, pl.ds(sbase, CH)],
                                       wsem.at[k]))

        def mean5(j, _):
            for c in range(D // 16):
                s = pl.ds(c * 16, 16)
                acc = (buf[j, s] + buf[CH + j, s] + buf[2 * CH + j, s]
                       + buf[3 * CH + j, s] + buf[4 * CH + j, s])
                obuf[j, s] = acc * (1.0 / S1)
            return 0

        lax.fori_loop(0, CH, mean5, 0)
        if wNm is not None:
            wNm.wait()
        wNm = pltpu.async_copy(obuf, Nm0.at[t, pl.ds(sbase, CH)], nsem)
        for cp in ws:
            cp.wait()
    wNm.wait()

    units = [(t, k, h) for t in range(T) for k in range(S1) for h in range(2)]
    H = CH // 2
    ngroups = len(units) // S1
    for gi in range(ngroups):
        gu = units[gi * S1:(gi + 1) * S1]
        cps = [pltpu.async_copy(x.at[t].at[ip.at[16 + t * 2 * S1 + k * 2 + h]],
                                slot(s), gsem.at[s])
               for s, (t, k, h) in enumerate(gu)]
        ws = []
        for s, (t, k, h) in enumerate(gu):
            cps[s].wait()

            def mean2(j, _, base=s * CH):
                for c in range(D // 16):
                    sl = pl.ds(c * 16, 16)
                    buf[base + j, sl] = (buf[base + 2 * j, sl]
                                         + buf[base + 2 * j + 1, sl]) * 0.5
                return 0

            lax.fori_loop(0, H, mean2, 0)
            ws.append(pltpu.async_copy(
                slot(s, H), Nm1.at[t, k, pl.ds(sbase + h * H, H)],
                wsem.at[s]))
        for cp in ws:
            cp.wait()


@functools.lru_cache(maxsize=1)
def _make_gather_call():
    return functools.partial(
        pl.kernel,
        out_type=[
            jax.ShapeDtypeStruct((T, B, D), jnp.float32),
            jax.ShapeDtypeStruct((T, S1, B, D), jnp.float32),
            jax.ShapeDtypeStruct((T, B, D), jnp.float32),
            jax.ShapeDtypeStruct((T, S1, B, D), jnp.float32),
        ],
        mesh=plsc.VectorSubcoreMesh(core_axis_name="c", subcore_axis_name="s",
                                    num_cores=NC, num_subcores=NS),
        compiler_params=pltpu.CompilerParams(use_tc_tiling_on_sc=True,
                                             needs_layout_passes=False),
        scratch_types=[
            pltpu.VMEM((S1 * CH, D), jnp.float32),
            pltpu.VMEM((CH, D), jnp.float32),
            pltpu.VMEM((46, CH), jnp.int32),
            pltpu.VMEM((T * S1 * CH,), jnp.int32),
            pltpu.VMEM((T * S1 * S2 * CH,), jnp.int32),
            pltpu.SemaphoreType.DMA((S1,)),
            pltpu.SemaphoreType.DMA((S1,)),
            pltpu.SemaphoreType.DMA,
        ],
    )(_gather_body)


def _dense_body(a0, a1, m0, m1, ws0, wn0, b0, ws1, wn1, b1, wp, bp, out):
    f32 = jnp.float32
    s1s = []
    for t in range(T):
        p0 = (jnp.dot(a0[t], ws0[...], preferred_element_type=f32)
              + jnp.dot(m0[t], wn0[...], preferred_element_type=f32) + b0[...])
        g0 = (p0 >= V_TH).astype(f32)
        nn = jnp.zeros((a0.shape[1], H0), f32)
        for k in range(S1):
            p1 = (jnp.dot(a1[t, k], ws0[...], preferred_element_type=f32)
                  + jnp.dot(m1[t, k], wn0[...], preferred_element_type=f32)
                  + b0[...])
            nn = nn + (p1 >= V_TH).astype(f32)
        q = (jnp.dot(g0, ws1[...], preferred_element_type=f32)
             + jnp.dot(nn * (1.0 / S1), wn1[...], preferred_element_type=f32)
             + b1[...])
        s1s.append((q >= V_TH).astype(f32))
    sp = jnp.concatenate(s1s, axis=1)
    out[...] = jnp.dot(sp, wp[...], preferred_element_type=f32) + bp[...]


def _dense_stage(a0, a1, m0, m1, ws0, wn0, b0, ws1, wn1, b1, wp, bp):
    blk = 256
    grid = (B // blk,)
    full = lambda shape: pl.BlockSpec(shape, lambda i: (0,) * len(shape))
    return pl.pallas_call(
        _dense_body,
        grid=grid,
        in_specs=[
            pl.BlockSpec((T, blk, D), lambda i: (0, i, 0)),
            pl.BlockSpec((T, S1, blk, D), lambda i: (0, 0, i, 0)),
            pl.BlockSpec((T, blk, D), lambda i: (0, i, 0)),
            pl.BlockSpec((T, S1, blk, D), lambda i: (0, 0, i, 0)),
            full((D, H0)), full((D, H0)), full((1, H0)),
            full((H0, H1)), full((H0, H1)), full((1, H1)),
            full((T * H1, OUT)), full((1, OUT)),
        ],
        out_specs=pl.BlockSpec((blk, OUT), lambda i: (i, 0)),
        out_shape=jax.ShapeDtypeStruct((B, OUT), jnp.float32),
    )(a0, a1, m0, m1, ws0, wn0, b0, ws1, wn1, b1, wp, bp)


def kernel(x, nodes, nbr1, nbr2, W_self0, W_neigh0, b0, W_self1, W_neigh1,
           b1, Wp, bp):
    A0, A1, Nm0, Nm1 = _make_gather_call()(
        x, nodes.astype(jnp.int32), nbr1.astype(jnp.int32),
        nbr2.astype(jnp.int32))

    return _dense_stage(
        A0, A1, Nm0, Nm1,
        W_self0, W_neigh0, b0.reshape(1, H0),
        W_self1, W_neigh1, b1.reshape(1, H1),
        Wp, bp.reshape(1, OUT))


# speedup vs baseline: 4.6995x; 1.1626x over previous
import functools

import jax
import jax.numpy as jnp
from jax import lax
from jax.experimental import pallas as pl
from jax.experimental.pallas import tpu as pltpu
from jax.experimental.pallas import tpu_sc as plsc

T = 3
N = 100000
D = 128
B = 4096
S1, S2 = 5, 2
H0, H1 = 64, 32
OUT = 64
V_TH = 1.0

NC, NS = 2, 16
NW = NC * NS
CH = 128

A0_TASKS = (T * B) // (NW * CH)
A1_TASKS = (T * B) // (NW * CH)
N1_TASKS = (T * S1 * B) // (NW * CH)


def _gather_body(x, nodes, nbr1, nbr2, A0, A1, Nm0, Nm1,
                 buf, obuf, ip, w1, w2, gsem, wsem, nsem):
    wid = lax.axis_index("s") * NC + lax.axis_index("c")
    lanes = jnp.arange(16, dtype=jnp.int32)
    G = CH // 16
    sbase = wid * CH

    def slot(s, n=CH):
        return buf.at[pl.ds(s * CH, n)]

    pltpu.sync_copy(nodes.at[pl.ds(sbase, CH)], ip.at[0])
    gA0 = [pltpu.async_copy(x.at[t].at[ip.at[0]], slot(t), gsem.at[t])
           for t in range(T)]
    def wcopy(t, _):
        pltpu.sync_copy(nbr1.at[t, pl.ds(sbase * S1, CH * S1)],
                        w1.at[pl.ds(t * CH * S1, CH * S1)])
        pltpu.sync_copy(nbr2.at[t, pl.ds(sbase * S1 * S2, CH * S1 * S2)],
                        w2.at[pl.ds(t * CH * S1 * S2, CH * S1 * S2)])
        return 0

    lax.fori_loop(0, T, wcopy, 0)

    def ex1(m, _):
        q = m // G
        t, k, g = q // S1, q % S1, m % G
        iv = lanes * S1 + (t * CH * S1 + g * 16 * S1 + k)
        ip[1 + q, pl.ds((m % G) * 16, 16)] = plsc.load_gather(w1, [iv])
        return 0

    lax.fori_loop(0, T * S1 * G, ex1, 0)

    def ex2(m, _):
        q = m // G
        r, g = q % (2 * S1), m % G
        t, k, h = q // (2 * S1), r // 2, r % 2
        iv = ((lanes // 2) * (S1 * S2) + (lanes % 2)
              + (t * CH * S1 * S2 + h * (CH * S1 * S2 // 2)
                 + g * 8 * S1 * S2 + k * S2))
        ip[16 + q, pl.ds(g * 16, 16)] = plsc.load_gather(w2, [iv])
        return 0

    lax.fori_loop(0, T * 2 * S1 * G, ex2, 0)

    H = CH // 2
    units = [("a0", t, 0, 0) for t in range(T)]
    for t in range(T):
        units += [("a1", t, k, 0) for k in range(S1)]
        units += [("n1", t, k, h) for k in range(S1) for h in range(2)]
    NU = len(units)
    F = 4

    def fire(u):
        ty, t, k, h = units[u]
        s = u % S1
        if ty == "a0":
            row = ip.at[0]
        elif ty == "a1":
            row = ip.at[1 + t * S1 + k]
        else:
            row = ip.at[16 + t * 2 * S1 + k * 2 + h]
        return pltpu.async_copy(x.at[t].at[row], slot(s), gsem.at[s])

    gps = list(gA0) + [fire(3)]
    wps = [None] * NU
    wNm = None
    for u in range(NU):
        ty, t, k, h = units[u]
        s = u % S1
        gps[u].wait()
        if ty == "a0":
            wps[u] = [pltpu.async_copy(slot(s), A0.at[t, pl.ds(sbase, CH)],
                                       wsem.at[s])]
        elif ty == "a1":
            base = s * CH

            def accum(j, _, base=base, k=k):
                for c in range(D // 16):
                    sl = pl.ds(c * 16, 16)
                    if k == 0:
                        obuf[j, sl] = buf[base + j, sl]
                    elif k < S1 - 1:
                        obuf[j, sl] = obuf[j, sl] + buf[base + j, sl]
                    else:
                        obuf[j, sl] = ((obuf[j, sl] + buf[base + j, sl])
                                       * (1.0 / S1))
                return 0

            if k == 0 and wNm is not None:
                wNm.wait()
            lax.fori_loop(0, CH, accum, 0)
            wps[u] = [pltpu.async_copy(slot(s), A1.at[t, k, pl.ds(sbase, CH)],
                                       wsem.at[s])]
            if k == S1 - 1:
                wNm = pltpu.async_copy(obuf, Nm0.at[t, pl.ds(sbase, CH)],
                                       nsem)
        else:
            base = s * CH

            def mean2(j, _, base=base):
                for c in range(D // 16):
                    sl = pl.ds(c * 16, 16)
                    buf[base + j, sl] = (buf[base + 2 * j, sl]
                                         + buf[base + 2 * j + 1, sl]) * 0.5
                return 0

            lax.fori_loop(0, H, mean2, 0)
            wps[u] = [pltpu.async_copy(slot(s, H),
                                       Nm1.at[t, k, pl.ds(sbase + h * H, H)],
                                       wsem.at[s])]
        nxt = u + F
        if nxt < NU:
            prev = nxt - S1
            if prev >= 0:
                for cp in wps[prev]:
                    cp.wait()
            gps.append(fire(nxt))
    for u in range(NU - S1, NU):
        for cp in wps[u]:
            cp.wait()
    wNm.wait()


@functools.lru_cache(maxsize=1)
def _make_gather_call():
    return functools.partial(
        pl.kernel,
        out_type=[
            jax.ShapeDtypeStruct((T, B, D), jnp.float32),
            jax.ShapeDtypeStruct((T, S1, B, D), jnp.float32),
            jax.ShapeDtypeStruct((T, B, D), jnp.float32),
            jax.ShapeDtypeStruct((T, S1, B, D), jnp.float32),
        ],
        mesh=plsc.VectorSubcoreMesh(core_axis_name="c", subcore_axis_name="s",
                                    num_cores=NC, num_subcores=NS),
        compiler_params=pltpu.CompilerParams(use_tc_tiling_on_sc=True,
                                             needs_layout_passes=False),
        scratch_types=[
            pltpu.VMEM((S1 * CH, D), jnp.float32),
            pltpu.VMEM((CH, D), jnp.float32),
            pltpu.VMEM((46, CH), jnp.int32),
            pltpu.VMEM((T * S1 * CH,), jnp.int32),
            pltpu.VMEM((T * S1 * S2 * CH,), jnp.int32),
            pltpu.SemaphoreType.DMA((S1,)),
            pltpu.SemaphoreType.DMA((S1,)),
            pltpu.SemaphoreType.DMA,
        ],
    )(_gather_body)


def _dense_body(a0, a1, m0, m1, ws0, wn0, b0, ws1, wn1, b1, wp, bp, out):
    f32 = jnp.float32
    s1s = []
    for t in range(T):
        p0 = (jnp.dot(a0[t], ws0[...], preferred_element_type=f32)
              + jnp.dot(m0[t], wn0[...], preferred_element_type=f32) + b0[...])
        g0 = (p0 >= V_TH).astype(f32)
        nn = jnp.zeros((a0.shape[1], H0), f32)
        for k in range(S1):
            p1 = (jnp.dot(a1[t, k], ws0[...], preferred_element_type=f32)
                  + jnp.dot(m1[t, k], wn0[...], preferred_element_type=f32)
                  + b0[...])
            nn = nn + (p1 >= V_TH).astype(f32)
        q = (jnp.dot(g0, ws1[...], preferred_element_type=f32)
             + jnp.dot(nn * (1.0 / S1), wn1[...], preferred_element_type=f32)
             + b1[...])
        s1s.append((q >= V_TH).astype(f32))
    sp = jnp.concatenate(s1s, axis=1)
    out[...] = jnp.dot(sp, wp[...], preferred_element_type=f32) + bp[...]


def _dense_stage(a0, a1, m0, m1, ws0, wn0, b0, ws1, wn1, b1, wp, bp):
    blk = 256
    grid = (B // blk,)
    full = lambda shape: pl.BlockSpec(shape, lambda i: (0,) * len(shape))
    return pl.pallas_call(
        _dense_body,
        grid=grid,
        in_specs=[
            pl.BlockSpec((T, blk, D), lambda i: (0, i, 0)),
            pl.BlockSpec((T, S1, blk, D), lambda i: (0, 0, i, 0)),
            pl.BlockSpec((T, blk, D), lambda i: (0, i, 0)),
            pl.BlockSpec((T, S1, blk, D), lambda i: (0, 0, i, 0)),
            full((D, H0)), full((D, H0)), full((1, H0)),
            full((H0, H1)), full((H0, H1)), full((1, H1)),
            full((T * H1, OUT)), full((1, OUT)),
        ],
        out_specs=pl.BlockSpec((blk, OUT), lambda i: (i, 0)),
        out_shape=jax.ShapeDtypeStruct((B, OUT), jnp.float32),
    )(a0, a1, m0, m1, ws0, wn0, b0, ws1, wn1, b1, wp, bp)


def kernel(x, nodes, nbr1, nbr2, W_self0, W_neigh0, b0, W_self1, W_neigh1,
           b1, Wp, bp):
    A0, A1, Nm0, Nm1 = _make_gather_call()(
        x, nodes.astype(jnp.int32), nbr1.astype(jnp.int32),
        nbr2.astype(jnp.int32))

    return _dense_stage(
        A0, A1, Nm0, Nm1,
        W_self0, W_neigh0, b0.reshape(1, H0),
        W_self1, W_neigh1, b1.reshape(1, H1),
        Wp, bp.reshape(1, OUT))
